# baseline (speedup 1.0000x reference)
import jax
import jax.numpy as jnp
from jax.experimental import pallas as pl
from jax.experimental.pallas import tpu as pltpu
from jax.experimental.pallas import tpu_sc as plsc

_W = 256


def _lookup(t128, idx_flat, embed):
    n = idx_flat.shape[0]
    n_steps = n // _W

    mesh = plsc.VectorSubcoreMesh(core_axis_name="core",
                                  subcore_axis_name="subcore")

    @pl.kernel(
        out_type=jax.ShapeDtypeStruct((n, embed), jnp.float32),
        mesh=mesh,
        compiler_params=pltpu.CompilerParams(use_tc_tiling_on_sc=True),
        scratch_types=[
            pltpu.VMEM((_W, 128), jnp.float32),
            pltpu.SemaphoreType.DMA,
        ],
    )
    def lookup(t128_hbm, idx_hbm, out_hbm, rows_vmem, sem):
        def body(idx_vmem, out_vmem):
            pltpu.async_copy(t128_hbm.at[idx_vmem], rows_vmem, sem).wait()

            @pl.loop(0, _W)
            def _(j):
                for k in range(embed // 16):
                    out_vmem[j, pl.ds(16 * k, 16)] = (
                        rows_vmem[j, pl.ds(16 * k, 16)])

        pltpu.emit_pipeline(
            body,
            grid=(n_steps,),
            in_specs=[pl.BlockSpec((_W,), index_map=lambda i: (i,))],
            out_specs=[pl.BlockSpec((_W, embed), index_map=lambda i: (i, 0))],
            core_axis_name=("core", "subcore"),
            dimension_semantics=(pltpu.PARALLEL,),
        )(idx_hbm, out_hbm)

    return lookup(t128, idx_flat)


def kernel(input, table):
    batch, seq = input.shape
    vocab, embed = table.shape
    t128 = jnp.pad(table, ((0, 0), (0, 128 - embed)))
    idx_flat = input.reshape(-1).astype(jnp.int32)
    out = _lookup(t128, idx_flat, embed)
    return out.reshape(batch, seq, embed)


# speedup vs baseline: 1.2805x; 1.2805x over previous
import jax
import jax.numpy as jnp
from jax import lax
from jax.experimental import pallas as pl
from jax.experimental.pallas import tpu as pltpu
from jax.experimental.pallas import tpu_sc as plsc

_W = 128
_NW = 32


def _lookup(t128, idx_flat, embed):
    n = idx_flat.shape[0]
    per_sub = n // _NW
    n_win = per_sub // _W

    mesh = plsc.VectorSubcoreMesh(core_axis_name="core",
                                  subcore_axis_name="subcore")

    @pl.kernel(
        out_type=jax.ShapeDtypeStruct((n, embed), jnp.float32),
        mesh=mesh,
        compiler_params=pltpu.CompilerParams(use_tc_tiling_on_sc=True),
        scratch_types=[
            pltpu.VMEM((per_sub,), jnp.int32),
            pltpu.VMEM((2, _W, 128), jnp.float32),
            pltpu.VMEM((2, _W, embed), jnp.float32),
            pltpu.SemaphoreType.DMA,
            pltpu.SemaphoreType.DMA,
            pltpu.SemaphoreType.DMA,
        ],
    )
    def lookup(t128_hbm, idx_hbm, out_hbm, idx_v, rows_v, cmp_v,
               gsem, osem0, osem1):
        wid = lax.axis_index("subcore") * 2 + lax.axis_index("core")
        base = wid * per_sub

        pltpu.sync_copy(idx_hbm.at[pl.ds(base, per_sub)], idx_v)
        pltpu.async_copy(t128_hbm.at[idx_v.at[pl.ds(0, _W)]],
                         rows_v.at[0], gsem)

        def compact(src, dst):
            @pl.loop(0, _W)
            def _(j):
                for k in range(embed // 16):
                    dst[j, pl.ds(16 * k, 16)] = src[j, pl.ds(16 * k, 16)]

        def step(w, slot):
            pltpu.make_async_copy(t128_hbm.at[idx_v.at[pl.ds(0, _W)]],
                                  rows_v.at[slot], gsem).wait()

            @pl.when(w + 1 < n_win)
            def _():
                pltpu.async_copy(
                    t128_hbm.at[idx_v.at[pl.ds((w + 1) * _W, _W)]],
                    rows_v.at[1 - slot], gsem)

            osem = osem0 if slot == 0 else osem1

            @pl.when(w >= 2)
            def _():
                pltpu.make_async_copy(
                    cmp_v.at[slot],
                    out_hbm.at[pl.ds(base + (w - 2) * _W, _W)], osem).wait()

            compact(rows_v.at[slot], cmp_v.at[slot])
            pltpu.async_copy(cmp_v.at[slot],
                             out_hbm.at[pl.ds(base + w * _W, _W)], osem)

        @pl.loop(0, n_win // 2)
        def _(h):
            step(2 * h, 0)
            step(2 * h + 1, 1)

        pltpu.make_async_copy(
            cmp_v.at[0], out_hbm.at[pl.ds(base + (n_win - 2) * _W, _W)],
            osem0).wait()
        pltpu.make_async_copy(
            cmp_v.at[1], out_hbm.at[pl.ds(base + (n_win - 1) * _W, _W)],
            osem1).wait()

    return lookup(t128, idx_flat)


def kernel(input, table):
    batch, seq = input.shape
    vocab, embed = table.shape
    t128 = jnp.pad(table, ((0, 0), (0, 128 - embed)))
    idx_flat = input.reshape(-1).astype(jnp.int32)
    out = _lookup(t128, idx_flat, embed)
    return out.reshape(batch, seq, embed)
